# baseline (speedup 1.0000x reference)
import functools

import jax
import jax.numpy as jnp
from jax import lax
from jax.experimental import pallas as pl
from jax.experimental.pallas import tpu as pltpu
from jax.experimental.pallas import tpu_sc as plsc

N = 10000
E = 160000
IN_DIM = 256
HID = 256
NUM_CLASS = 64
NEG_SLOPE = 0.2

NC, NS = 2, 16
EK = 128
E_PAD = 163840
DUMMY = N
S_ROWS = N + 8
ROWS_PER_TILE = N // NS
CHUNKS = E_PAD // (NS * EK)
DEG_CHUNKS = CHUNKS // NC

_MESH = plsc.VectorSubcoreMesh(
    core_axis_name="c", subcore_axis_name="s", num_cores=NC, num_subcores=NS
)
_SC_PARAMS = pltpu.CompilerParams(use_tc_tiling_on_sc=False)


@functools.partial(
    pl.kernel,
    out_type=jax.ShapeDtypeStruct((NC, N, 16), jnp.float32),
    mesh=_MESH,
    scratch_types=[
        pltpu.VMEM((DEG_CHUNKS, 2, EK), jnp.int32),
        pltpu.VMEM((EK, 16), jnp.float32),
        pltpu.VMEM_SHARED((S_ROWS, 16), jnp.float32),
    ],
    compiler_params=_SC_PARAMS,
)
def _deg_kernel(idx_hbm, ones_hbm, zeros_hbm, out_hbm, idx_all, ones_v, acc):
    cid = lax.axis_index("c")
    sid = lax.axis_index("s")
    r0 = sid * ROWS_PER_TILE
    pltpu.sync_copy(zeros_hbm.at[pl.ds(r0, ROWS_PER_TILE)],
                    acc.at[pl.ds(r0, ROWS_PER_TILE)])
    pltpu.sync_copy(ones_hbm, ones_v)
    pltpu.sync_copy(idx_hbm.at[sid, pl.ds(cid * DEG_CHUNKS, DEG_CHUNKS)],
                    idx_all)
    plsc.subcore_barrier()

    def body(j, carry):
        pltpu.sync_copy(ones_v, acc.at[idx_all.at[j, 1]], add=True)
        return carry

    lax.fori_loop(0, DEG_CHUNKS, body, 0)
    plsc.subcore_barrier()
    pltpu.sync_copy(acc.at[pl.ds(r0, ROWS_PER_TILE)],
                    out_hbm.at[cid, pl.ds(r0, ROWS_PER_TILE)])


def _make_spmm(width):
    @functools.partial(
        pl.kernel,
        out_type=(
            jax.ShapeDtypeStruct((N, width), jnp.float32),
            jax.ShapeDtypeStruct((N, width), jnp.float32),
        ),
        mesh=_MESH,
        scratch_types=(
            [pltpu.VMEM((2, EK), jnp.int32) for _ in range(3)]
            + [pltpu.VMEM((EK, width), jnp.float32) for _ in range(3)]
            + [pltpu.VMEM_SHARED((S_ROWS, width), jnp.float32)]
            + [pltpu.SemaphoreType.DMA for _ in range(9)]
        ),
        compiler_params=_SC_PARAMS,
    )
    def spmm(idx_hbm, pa_hbm, pb_hbm, oa_hbm, ob_hbm,
             i0, i1, i2, w0, w1, w2, acc,
             si0, si1, si2, sg0, sg1, sg2, ss0, ss1, ss2):
        cid = lax.axis_index("c")
        sid = lax.axis_index("s")
        r0 = sid * ROWS_PER_TILE
        IB = (i0, i1, i2)
        RW = (w0, w1, w2)
        SI = (si0, si1, si2)
        SG = (sg0, sg1, sg2)
        SS = (ss0, ss1, ss2)

        def run(p_hbm, o_hbm):
            def start_i(j, b):
                pltpu.async_copy(idx_hbm.at[sid, j], IB[b], SI[b])

            def wait_i(j, b):
                pltpu.make_async_copy(idx_hbm.at[sid, j], IB[b], SI[b]).wait()

            def start_g(b):
                pltpu.async_copy(p_hbm.at[IB[b].at[0]], RW[b], SG[b])

            def wait_g(b):
                pltpu.make_async_copy(p_hbm.at[IB[b].at[0]], RW[b],
                                      SG[b]).wait()

            def start_s(b):
                pltpu.async_copy(RW[b], acc.at[IB[b].at[1]], SS[b], add=True)

            def wait_s(b):
                pltpu.make_async_copy(RW[b], acc.at[IB[b].at[1]],
                                      SS[b]).wait()

            def step(j, b, b1, b2, do_wait_s, do_i, do_g):
                wait_g(b)
                start_s(b)
                if do_wait_s:
                    wait_s(b1)
                if do_i:
                    start_i(j + 2, b1)
                if do_g:
                    wait_i(j + 1, b2)
                    start_g(b2)

            pltpu.sync_copy(p_hbm.at[pl.ds(r0, ROWS_PER_TILE)],
                            acc.at[pl.ds(r0, ROWS_PER_TILE)])
            plsc.subcore_barrier()

            start_i(0, 0)
            start_i(1, 1)
            wait_i(0, 0)
            start_g(0)
            step(0, 0, 2, 1, False, True, True)

            n_main = ((CHUNKS - 5) // 3) * 3

            def body(g, carry):
                for o in range(3):
                    j = 3 * g + 1 + o
                    step(j, (1 + o) % 3, o % 3, (2 + o) % 3, True, True, True)
                return carry

            lax.fori_loop(0, n_main // 3, body, 0)
            for j in range(n_main + 1, CHUNKS):
                step(j, j % 3, (j - 1) % 3, (j + 1) % 3,
                     True, j + 2 < CHUNKS, j + 1 < CHUNKS)
            wait_s((CHUNKS - 1) % 3)
            plsc.subcore_barrier()
            pltpu.sync_copy(acc.at[pl.ds(r0, ROWS_PER_TILE)],
                            o_hbm.at[pl.ds(r0, ROWS_PER_TILE)])

        @pl.when(cid == 0)
        def _():
            run(pa_hbm, oa_hbm)

        @pl.when(cid == 1)
        def _():
            run(pb_hbm, ob_hbm)

    return spmm


_spmm_l1 = _make_spmm(HID // 2)
_spmm_l2 = _make_spmm(NUM_CLASS // 2)


_R = 1000


def _tc_layer1(x, W1, cnt0, cnt1):
    def body(x_ref, w_ref, c0_ref, c1_ref, pa_ref, pb_ref, s_ref):
        s = lax.rsqrt(c0_ref[...] + c1_ref[...] + 1.0)
        p = jnp.dot(x_ref[...], w_ref[...],
                    preferred_element_type=jnp.float32) * s
        pa_ref[...] = p[:, : HID // 2]
        pb_ref[...] = p[:, HID // 2:]
        s_ref[...] = s

    return pl.pallas_call(
        body,
        grid=(N // _R,),
        in_specs=[
            pl.BlockSpec((_R, IN_DIM), lambda i: (i, 0)),
            pl.BlockSpec((IN_DIM, HID), lambda i: (0, 0)),
            pl.BlockSpec((_R, 1), lambda i: (i, 0)),
            pl.BlockSpec((_R, 1), lambda i: (i, 0)),
        ],
        out_specs=[
            pl.BlockSpec((_R, HID // 2), lambda i: (i, 0)),
            pl.BlockSpec((_R, HID // 2), lambda i: (i, 0)),
            pl.BlockSpec((_R, 1), lambda i: (i, 0)),
        ],
        out_shape=[
            jax.ShapeDtypeStruct((N, HID // 2), jnp.float32),
            jax.ShapeDtypeStruct((N, HID // 2), jnp.float32),
            jax.ShapeDtypeStruct((N, 1), jnp.float32),
        ],
    )(x, W1, cnt0, cnt1)


def _tc_mid(aa, ab, s, b1, W2):
    def body(aa_ref, ab_ref, s_ref, b1_ref, w2_ref, pa_ref, pb_ref):
        sv = s_ref[...]
        h = jnp.concatenate([aa_ref[...], ab_ref[...]], axis=1) * sv + b1_ref[...]
        h = jnp.where(h >= 0, h, NEG_SLOPE * h)
        p = jnp.dot(h, w2_ref[...], preferred_element_type=jnp.float32) * sv
        pa_ref[...] = p[:, : NUM_CLASS // 2]
        pb_ref[...] = p[:, NUM_CLASS // 2:]

    return pl.pallas_call(
        body,
        grid=(N // _R,),
        in_specs=[
            pl.BlockSpec((_R, HID // 2), lambda i: (i, 0)),
            pl.BlockSpec((_R, HID // 2), lambda i: (i, 0)),
            pl.BlockSpec((_R, 1), lambda i: (i, 0)),
            pl.BlockSpec((1, HID), lambda i: (0, 0)),
            pl.BlockSpec((HID, NUM_CLASS), lambda i: (0, 0)),
        ],
        out_specs=[
            pl.BlockSpec((_R, NUM_CLASS // 2), lambda i: (i, 0)),
            pl.BlockSpec((_R, NUM_CLASS // 2), lambda i: (i, 0)),
        ],
        out_shape=[
            jax.ShapeDtypeStruct((N, NUM_CLASS // 2), jnp.float32),
            jax.ShapeDtypeStruct((N, NUM_CLASS // 2), jnp.float32),
        ],
    )(aa, ab, s, b1, W2)


def _tc_out(aa, ab, s, b2):
    def body(aa_ref, ab_ref, s_ref, b2_ref, o_ref):
        z = jnp.concatenate([aa_ref[...], ab_ref[...]], axis=1) * s_ref[...]
        z = z + b2_ref[...]
        m = jnp.max(z, axis=1, keepdims=True)
        e = z - m
        o_ref[...] = e - jnp.log(jnp.sum(jnp.exp(e), axis=1, keepdims=True))

    return pl.pallas_call(
        body,
        grid=(N // _R,),
        in_specs=[
            pl.BlockSpec((_R, NUM_CLASS // 2), lambda i: (i, 0)),
            pl.BlockSpec((_R, NUM_CLASS // 2), lambda i: (i, 0)),
            pl.BlockSpec((_R, 1), lambda i: (i, 0)),
            pl.BlockSpec((1, NUM_CLASS), lambda i: (0, 0)),
        ],
        out_specs=pl.BlockSpec((_R, NUM_CLASS), lambda i: (i, 0)),
        out_shape=jax.ShapeDtypeStruct((N, NUM_CLASS), jnp.float32),
    )(aa, ab, s, b2)


def kernel(x, edge_index, W1, b1, W2, b2):
    src = edge_index[0].astype(jnp.int32)
    dst = edge_index[1].astype(jnp.int32)
    pad = E_PAD - E
    src_p = jnp.concatenate([src, jnp.zeros((pad,), jnp.int32)])
    dst_p = jnp.concatenate([dst, jnp.full((pad,), DUMMY, jnp.int32)])
    idx = jnp.stack([src_p.reshape(NS, CHUNKS, EK),
                     dst_p.reshape(NS, CHUNKS, EK)], axis=2)
    ones = jnp.ones((EK, 16), jnp.float32)
    zeros = jnp.zeros((N, 16), jnp.float32)

    cnt = _deg_kernel(idx, ones, zeros)
    cnt0 = lax.slice(cnt, (0, 0, 0), (1, N, 1)).reshape(N, 1)
    cnt1 = lax.slice(cnt, (1, 0, 0), (2, N, 1)).reshape(N, 1)

    pa, pb, s = _tc_layer1(x, W1, cnt0, cnt1)
    agg_a, agg_b = _spmm_l1(idx, pa, pb)
    p2a, p2b = _tc_mid(agg_a, agg_b, s, b1.reshape(1, HID), W2)
    agg2a, agg2b = _spmm_l2(idx, p2a, p2b)
    return _tc_out(agg2a, agg2b, s, b2.reshape(1, NUM_CLASS))


# speedup vs baseline: 1.0951x; 1.0951x over previous
import functools

import jax
import jax.numpy as jnp
from jax import lax
from jax.experimental import pallas as pl
from jax.experimental.pallas import tpu as pltpu
from jax.experimental.pallas import tpu_sc as plsc

N = 10000
E = 160000
IN_DIM = 256
HID = 256
NUM_CLASS = 64
NEG_SLOPE = 0.2

NC, NS = 2, 16
EK = 128
E_PAD = 163840
DUMMY = N
S_ROWS = N + 8
ROWS_PER_TILE = N // NS
CHUNKS = E_PAD // (NS * EK)
DEG_CHUNKS = CHUNKS // NC

_MESH = plsc.VectorSubcoreMesh(
    core_axis_name="c", subcore_axis_name="s", num_cores=NC, num_subcores=NS
)
_SC_PARAMS = pltpu.CompilerParams(use_tc_tiling_on_sc=False)


@functools.partial(
    pl.kernel,
    out_type=jax.ShapeDtypeStruct((NC, N, 16), jnp.float32),
    mesh=_MESH,
    scratch_types=[
        pltpu.VMEM((DEG_CHUNKS, 2, EK), jnp.int32),
        pltpu.VMEM((EK, 16), jnp.float32),
        pltpu.VMEM_SHARED((S_ROWS, 16), jnp.float32),
    ],
    compiler_params=_SC_PARAMS,
)
def _deg_kernel(idx_hbm, ones_hbm, zeros_hbm, out_hbm, idx_all, ones_v, acc):
    cid = lax.axis_index("c")
    sid = lax.axis_index("s")
    r0 = sid * ROWS_PER_TILE
    pltpu.sync_copy(zeros_hbm.at[pl.ds(r0, ROWS_PER_TILE)],
                    acc.at[pl.ds(r0, ROWS_PER_TILE)])
    pltpu.sync_copy(ones_hbm, ones_v)
    pltpu.sync_copy(idx_hbm.at[sid, pl.ds(cid * DEG_CHUNKS, DEG_CHUNKS)],
                    idx_all)
    plsc.subcore_barrier()

    def body(j, carry):
        pltpu.sync_copy(ones_v, acc.at[idx_all.at[j, 1]], add=True)
        return carry

    lax.fori_loop(0, DEG_CHUNKS, body, 0)
    plsc.subcore_barrier()
    pltpu.sync_copy(acc.at[pl.ds(r0, ROWS_PER_TILE)],
                    out_hbm.at[cid, pl.ds(r0, ROWS_PER_TILE)])


def _make_spmm(width):
    @functools.partial(
        pl.kernel,
        out_type=(
            jax.ShapeDtypeStruct((N, width), jnp.float32),
            jax.ShapeDtypeStruct((N, width), jnp.float32),
        ),
        mesh=_MESH,
        scratch_types=(
            [pltpu.VMEM((2, EK), jnp.int32) for _ in range(4)]
            + [pltpu.VMEM((EK, width), jnp.float32) for _ in range(2)]
            + [pltpu.VMEM_SHARED((S_ROWS, width), jnp.float32)]
            + [pltpu.SemaphoreType.DMA for _ in range(6)]
        ),
        compiler_params=_SC_PARAMS,
    )
    def spmm(idx_hbm, pa_hbm, pb_hbm, oa_hbm, ob_hbm,
             i0, i1, i2, i3, w0, w1, acc,
             si0, si1, si2, si3, sg0, sg1):
        cid = lax.axis_index("c")
        sid = lax.axis_index("s")
        r0 = sid * ROWS_PER_TILE
        IB = (i0, i1, i2, i3)
        RW = (w0, w1)
        SI = (si0, si1, si2, si3)
        SG = (sg0, sg1)

        def run(p_hbm, o_hbm):
            def start_i(j, b):
                pltpu.async_copy(idx_hbm.at[sid, j], IB[b], SI[b])

            def wait_i(j, b):
                pltpu.make_async_copy(idx_hbm.at[sid, j], IB[b], SI[b]).wait()

            def start_g(bi, bw):
                pltpu.async_copy(p_hbm.at[IB[bi].at[0]], RW[bw], SG[bw])

            def wait_g(bi, bw):
                pltpu.make_async_copy(p_hbm.at[IB[bi].at[0]], RW[bw],
                                      SG[bw]).wait()

            def step(j, bi, bw, do_i, do_g):
                wait_g(bi, bw)
                pltpu.sync_copy(RW[bw], acc.at[IB[bi].at[1]], add=True)
                if do_i:
                    start_i(j + 4, bi)
                if do_g:
                    wait_i(j + 2, (bi + 2) % 4)
                    start_g((bi + 2) % 4, bw)

            pltpu.sync_copy(p_hbm.at[pl.ds(r0, ROWS_PER_TILE)],
                            acc.at[pl.ds(r0, ROWS_PER_TILE)])
            plsc.subcore_barrier()

            for t in range(4):
                start_i(t, t)
            wait_i(0, 0)
            start_g(0, 0)
            wait_i(1, 1)
            start_g(1, 1)
            step(0, 0, 0, True, True)

            n_main = ((CHUNKS - 5) // 4) * 4

            def body(g, carry):
                for o in range(4):
                    step(4 * g + 1 + o, (1 + o) % 4, (1 + o) % 2, True, True)
                return carry

            lax.fori_loop(0, n_main // 4, body, 0)
            for j in range(n_main + 1, CHUNKS):
                step(j, j % 4, j % 2, j + 4 < CHUNKS, j + 2 < CHUNKS)
            plsc.subcore_barrier()
            pltpu.sync_copy(acc.at[pl.ds(r0, ROWS_PER_TILE)],
                            o_hbm.at[pl.ds(r0, ROWS_PER_TILE)])

        @pl.when(cid == 0)
        def _():
            run(pa_hbm, oa_hbm)

        @pl.when(cid == 1)
        def _():
            run(pb_hbm, ob_hbm)

    return spmm


_spmm_l1 = _make_spmm(HID // 2)
_spmm_l2 = _make_spmm(NUM_CLASS // 2)


_R = 1000


def _tc_layer1(x, W1, cnt0, cnt1):
    def body(x_ref, w_ref, c0_ref, c1_ref, pa_ref, pb_ref, s_ref):
        s = lax.rsqrt(c0_ref[...] + c1_ref[...] + 1.0)
        p = jnp.dot(x_ref[...], w_ref[...],
                    preferred_element_type=jnp.float32) * s
        pa_ref[...] = p[:, : HID // 2]
        pb_ref[...] = p[:, HID // 2:]
        s_ref[...] = s

    return pl.pallas_call(
        body,
        grid=(N // _R,),
        in_specs=[
            pl.BlockSpec((_R, IN_DIM), lambda i: (i, 0)),
            pl.BlockSpec((IN_DIM, HID), lambda i: (0, 0)),
            pl.BlockSpec((_R, 1), lambda i: (i, 0)),
            pl.BlockSpec((_R, 1), lambda i: (i, 0)),
        ],
        out_specs=[
            pl.BlockSpec((_R, HID // 2), lambda i: (i, 0)),
            pl.BlockSpec((_R, HID // 2), lambda i: (i, 0)),
            pl.BlockSpec((_R, 1), lambda i: (i, 0)),
        ],
        out_shape=[
            jax.ShapeDtypeStruct((N, HID // 2), jnp.float32),
            jax.ShapeDtypeStruct((N, HID // 2), jnp.float32),
            jax.ShapeDtypeStruct((N, 1), jnp.float32),
        ],
    )(x, W1, cnt0, cnt1)


def _tc_mid(aa, ab, s, b1, W2):
    def body(aa_ref, ab_ref, s_ref, b1_ref, w2_ref, pa_ref, pb_ref):
        sv = s_ref[...]
        h = jnp.concatenate([aa_ref[...], ab_ref[...]], axis=1) * sv + b1_ref[...]
        h = jnp.where(h >= 0, h, NEG_SLOPE * h)
        p = jnp.dot(h, w2_ref[...], preferred_element_type=jnp.float32) * sv
        pa_ref[...] = p[:, : NUM_CLASS // 2]
        pb_ref[...] = p[:, NUM_CLASS // 2:]

    return pl.pallas_call(
        body,
        grid=(N // _R,),
        in_specs=[
            pl.BlockSpec((_R, HID // 2), lambda i: (i, 0)),
            pl.BlockSpec((_R, HID // 2), lambda i: (i, 0)),
            pl.BlockSpec((_R, 1), lambda i: (i, 0)),
            pl.BlockSpec((1, HID), lambda i: (0, 0)),
            pl.BlockSpec((HID, NUM_CLASS), lambda i: (0, 0)),
        ],
        out_specs=[
            pl.BlockSpec((_R, NUM_CLASS // 2), lambda i: (i, 0)),
            pl.BlockSpec((_R, NUM_CLASS // 2), lambda i: (i, 0)),
        ],
        out_shape=[
            jax.ShapeDtypeStruct((N, NUM_CLASS // 2), jnp.float32),
            jax.ShapeDtypeStruct((N, NUM_CLASS // 2), jnp.float32),
        ],
    )(aa, ab, s, b1, W2)


def _tc_out(aa, ab, s, b2):
    def body(aa_ref, ab_ref, s_ref, b2_ref, o_ref):
        z = jnp.concatenate([aa_ref[...], ab_ref[...]], axis=1) * s_ref[...]
        z = z + b2_ref[...]
        m = jnp.max(z, axis=1, keepdims=True)
        e = z - m
        o_ref[...] = e - jnp.log(jnp.sum(jnp.exp(e), axis=1, keepdims=True))

    return pl.pallas_call(
        body,
        grid=(N // _R,),
        in_specs=[
            pl.BlockSpec((_R, NUM_CLASS // 2), lambda i: (i, 0)),
            pl.BlockSpec((_R, NUM_CLASS // 2), lambda i: (i, 0)),
            pl.BlockSpec((_R, 1), lambda i: (i, 0)),
            pl.BlockSpec((1, NUM_CLASS), lambda i: (0, 0)),
        ],
        out_specs=pl.BlockSpec((_R, NUM_CLASS), lambda i: (i, 0)),
        out_shape=jax.ShapeDtypeStruct((N, NUM_CLASS), jnp.float32),
    )(aa, ab, s, b2)


def kernel(x, edge_index, W1, b1, W2, b2):
    src = edge_index[0].astype(jnp.int32)
    dst = edge_index[1].astype(jnp.int32)
    pad = E_PAD - E
    src_p = jnp.concatenate([src, jnp.zeros((pad,), jnp.int32)])
    dst_p = jnp.concatenate([dst, jnp.full((pad,), DUMMY, jnp.int32)])
    idx = jnp.stack([src_p.reshape(NS, CHUNKS, EK),
                     dst_p.reshape(NS, CHUNKS, EK)], axis=2)
    ones = jnp.ones((EK, 16), jnp.float32)
    zeros = jnp.zeros((N, 16), jnp.float32)

    cnt = _deg_kernel(idx, ones, zeros)
    cnt0 = lax.slice(cnt, (0, 0, 0), (1, N, 1)).reshape(N, 1)
    cnt1 = lax.slice(cnt, (1, 0, 0), (2, N, 1)).reshape(N, 1)

    pa, pb, s = _tc_layer1(x, W1, cnt0, cnt1)
    agg_a, agg_b = _spmm_l1(idx, pa, pb)
    p2a, p2b = _tc_mid(agg_a, agg_b, s, b1.reshape(1, HID), W2)
    agg2a, agg2b = _spmm_l2(idx, p2a, p2b)
    return _tc_out(agg2a, agg2b, s, b2.reshape(1, NUM_CLASS))
